# baseline (speedup 1.0000x reference)
import functools

import jax
import jax.numpy as jnp
from jax import lax
from jax.experimental import pallas as pl
from jax.experimental.pallas import tpu as pltpu
from jax.experimental.pallas import tpu_sc as plsc

_CTX = 50
_DIM = 32
_BATCH = 4096

_NC = 2
_NS = 16
_NW = _NC * _NS

_RPW = _BATCH // _NW
_NGRP = _RPW // 16
_FEAT = _CTX * _DIM
_DEPTH = 4


def _sc_body(ids_ref, table_ref, ws_ref, b_ref, out_ref,
             idx_v, r0, r1, r2, r3, ws_v, b_v, out_v, s0, s1, s2, s3):
    bufs = (r0, r1, r2, r3)
    sems = (s0, s1, s2, s3)
    wid = lax.axis_index("s") * _NC + lax.axis_index("c")
    col0 = wid * _RPW

    pltpu.sync_copy(ids_ref.at[:, pl.ds(col0, _RPW)], idx_v)
    pltpu.sync_copy(ws_ref, ws_v)
    pltpu.sync_copy(b_ref, b_v)
    b_vec = b_v[pl.ds(0, 16)]
    lanes = lax.iota(jnp.int32, 16)
    rgs = [lanes + 16 * g for g in range(_NGRP)]

    def fire(c, p):
        return pltpu.async_copy(table_ref.at[idx_v.at[c]], bufs[p], sems[p])

    def wait(c, p):
        pltpu.make_async_copy(table_ref.at[idx_v.at[c]], bufs[p],
                              sems[p]).wait()

    def compute(c, p, accs):
        buf = bufs[p]
        accs = list(accs)
        for m in range(_DIM):
            wv = ws_v[c * _DIM + m, pl.ds(0, 16)]
            cm = jnp.full((16,), m, jnp.int32)
            for g in range(_NGRP):
                accs[g] = accs[g] + plsc.load_gather(buf, [rgs[g], cm]) * wv
        return tuple(accs)

    for p in range(_DEPTH):
        fire(p, p)

    def loop_body(k, accs):
        for p in range(_DEPTH):
            c = _DEPTH * k + p
            wait(c, p)
            accs = compute(c, p, accs)

            @pl.when(c + _DEPTH < _CTX)
            def _():
                fire(c + _DEPTH, p)

        return accs

    accs = lax.fori_loop(0, _CTX // _DEPTH, loop_body,
                         tuple(b_vec for _ in range(_NGRP)))

    for p in range(_CTX % _DEPTH):
        c = _CTX - (_CTX % _DEPTH) + p
        wait(c, p)
        accs = compute(c, p, accs)

    for g in range(_NGRP):
        out_v[pl.ds(16 * g, 16)] = accs[g]
    pltpu.sync_copy(out_v, out_ref.at[pl.ds(col0, _RPW)])


@jax.jit
def _logits_sc(ids_t, table, w_splat, b16):
    mesh = plsc.VectorSubcoreMesh(
        core_axis_name="c", subcore_axis_name="s",
        num_cores=_NC, num_subcores=_NS)
    f = functools.partial(
        pl.kernel,
        out_type=jax.ShapeDtypeStruct((_BATCH,), jnp.float32),
        mesh=mesh,
        compiler_params=pltpu.CompilerParams(
            needs_layout_passes=False, use_tc_tiling_on_sc=False),
        scratch_types=[
            pltpu.VMEM((_CTX, _RPW), jnp.int32),
            pltpu.VMEM((_RPW, _DIM), jnp.float32),
            pltpu.VMEM((_RPW, _DIM), jnp.float32),
            pltpu.VMEM((_RPW, _DIM), jnp.float32),
            pltpu.VMEM((_RPW, _DIM), jnp.float32),
            pltpu.VMEM((_FEAT, 16), jnp.float32),
            pltpu.VMEM((16,), jnp.float32),
            pltpu.VMEM((_RPW,), jnp.float32),
            pltpu.SemaphoreType.DMA,
            pltpu.SemaphoreType.DMA,
            pltpu.SemaphoreType.DMA,
            pltpu.SemaphoreType.DMA,
        ],
    )(_sc_body)
    return f(ids_t, table, w_splat, b16)


def kernel(input_ids, table, W, b):
    ids_t = input_ids.astype(jnp.int32).T
    w_splat = jnp.broadcast_to(
        W.astype(jnp.float32).reshape(_FEAT, 1), (_FEAT, 16))
    b16 = jnp.broadcast_to(b.astype(jnp.float32), (16,))
    return _logits_sc(ids_t, table, w_splat, b16)
